# Initial kernel scaffold; baseline (speedup 1.0000x reference)
import functools

import jax
import jax.numpy as jnp
from jax import lax
from jax.experimental import pallas as pl
from jax.experimental.pallas import tpu as pltpu
from jax.experimental.pallas import tpu_sc as plsc

N = 10000
E = 320000
F = 128

NUM_CORES = 2
NUM_SUBCORES = 16
NUM_TILES = NUM_CORES * NUM_SUBCORES

EDGES_PER_TILE = E // NUM_TILES
CHUNK = 80
NCHUNKS = EDGES_PER_TILE // CHUNK
ROWS_PER_TILE = N // NUM_SUBCORES
ZROWS = 125

M_BLK = 1000


def _mm1_kernel(x_ref, w_ref, b_ref, o_ref):
    o_ref[...] = lax.dot_general(
        x_ref[...], w_ref[...], (((1,), (1,)), ((), ())),
        preferred_element_type=jnp.float32) + b_ref[...]


def _mm2_kernel(a_ref, w_ref, b_ref, o_ref):
    a = a_ref[0] + a_ref[1]
    o_ref[...] = lax.dot_general(
        a, w_ref[...], (((1,), (1,)), ((), ())),
        preferred_element_type=jnp.float32) + b_ref[...]


def _sc_body(h_hbm, src_hbm, dst_hbm, ew_hbm, out_hbm,
             sidx, didx, wbuf, rows, zbuf, acc, sem):
    c = lax.axis_index("c")
    s = lax.axis_index("s")
    wid = c * NUM_SUBCORES + s

    def _zero_body(i, _):
        z = jnp.zeros((16,), jnp.float32)
        for j in range(F // 16):
            zbuf[i, pl.ds(j * 16, 16)] = z
        return 0
    lax.fori_loop(0, ZROWS, _zero_body, 0)
    for t in range(ROWS_PER_TILE // ZROWS):
        pltpu.sync_copy(zbuf, acc.at[pl.ds(s * ROWS_PER_TILE + t * ZROWS, ZROWS)])
    plsc.subcore_barrier()

    def _edge_chunk(i, _):
        base = wid * EDGES_PER_TILE + i * CHUNK
        pltpu.sync_copy(src_hbm.at[pl.ds(base, CHUNK)], sidx)
        pltpu.sync_copy(dst_hbm.at[pl.ds(base, CHUNK)], didx)
        pltpu.sync_copy(ew_hbm.at[pl.ds(base, CHUNK)], wbuf)
        pltpu.async_copy(h_hbm.at[sidx], rows, sem).wait()

        def _scale(k, _):
            wk = wbuf[k]
            for j in range(F // 16):
                rows[k, pl.ds(j * 16, 16)] = rows[k, pl.ds(j * 16, 16)] * wk
            return 0
        lax.fori_loop(0, CHUNK, _scale, 0)

        pltpu.sync_copy(rows, acc.at[didx], add=True)
        return 0
    lax.fori_loop(0, NCHUNKS, _edge_chunk, 0)
    plsc.subcore_barrier()

    for t in range(ROWS_PER_TILE // ZROWS):
        r0 = s * ROWS_PER_TILE + t * ZROWS
        pltpu.sync_copy(acc.at[pl.ds(r0, ZROWS)], zbuf)
        pltpu.sync_copy(zbuf, out_hbm.at[c, pl.ds(r0, ZROWS)])


_sc_scatter = functools.partial(
    pl.kernel,
    mesh=plsc.VectorSubcoreMesh(core_axis_name="c", subcore_axis_name="s"),
    out_type=jax.ShapeDtypeStruct((NUM_CORES, N, F), jnp.float32),
    scratch_types=[
        pltpu.VMEM((CHUNK,), jnp.int32),
        pltpu.VMEM((CHUNK,), jnp.int32),
        pltpu.VMEM((CHUNK,), jnp.float32),
        pltpu.VMEM((CHUNK, F), jnp.float32),
        pltpu.VMEM((ZROWS, F), jnp.float32),
        pltpu.VMEM_SHARED((N, F), jnp.float32),
        pltpu.SemaphoreType.DMA,
    ],
)(_sc_body)


def kernel(x, edge_index, edge_weight, W_w, b_w, W_lin, b_lin):
    src = edge_index[0].astype(jnp.int32)
    dst = edge_index[1].astype(jnp.int32)
    ew = edge_weight.astype(jnp.float32)

    h = pl.pallas_call(
        _mm1_kernel,
        grid=(N // M_BLK,),
        in_specs=[
            pl.BlockSpec((M_BLK, F), lambda i: (i, 0)),
            pl.BlockSpec((F, F), lambda i: (0, 0)),
            pl.BlockSpec((1, F), lambda i: (0, 0)),
        ],
        out_specs=pl.BlockSpec((M_BLK, F), lambda i: (i, 0)),
        out_shape=jax.ShapeDtypeStruct((N, F), jnp.float32),
    )(x, W_w, b_w.reshape(1, F))

    partials = _sc_scatter(h, src, dst, ew)

    out = pl.pallas_call(
        _mm2_kernel,
        grid=(N // M_BLK,),
        in_specs=[
            pl.BlockSpec((NUM_CORES, M_BLK, F), lambda i: (0, i, 0)),
            pl.BlockSpec((F, F), lambda i: (0, 0)),
            pl.BlockSpec((1, F), lambda i: (0, 0)),
        ],
        out_specs=pl.BlockSpec((M_BLK, F), lambda i: (i, 0)),
        out_shape=jax.ShapeDtypeStruct((N, F), jnp.float32),
    )(partials, W_lin, b_lin.reshape(1, F))
    return out


# speedup vs baseline: 6.8665x; 6.8665x over previous
import functools

import numpy as np

import jax
import jax.numpy as jnp
from jax import lax
from jax.experimental import pallas as pl
from jax.experimental.pallas import tpu as pltpu
from jax.experimental.pallas import tpu_sc as plsc

N = 10000
E = 320000
F = 128

NUM_CORES = 2
NUM_SUBCORES = 16
NUM_TILES = NUM_CORES * NUM_SUBCORES

EDGES_PER_TILE = E // NUM_TILES
CHUNK = 80
NCHUNKS = EDGES_PER_TILE // CHUNK
NPAD = 10240
ROWS_PER_TILE = NPAD // NUM_SUBCORES

M_BLK = 1000

NBUF = 3
IDX_BITS = 14
IDX_MASK = (1 << IDX_BITS) - 1

_PERM = np.concatenate(
    [np.concatenate([32 * j + np.arange(0, 32, 2), 32 * j + np.arange(1, 32, 2)])
     for j in range(F // 32)])


def _mm1_kernel(x_ref, w_ref, b_ref, o_ref):
    o_ref[...] = lax.dot_general(
        x_ref[...], w_ref[...], (((1,), (1,)), ((), ())),
        preferred_element_type=jnp.float32) + b_ref[...]


def _mm2_kernel(a_ref, w_ref, b_ref, o_ref):
    a = a_ref[0] + a_ref[1]
    o_ref[...] = lax.dot_general(
        a, w_ref[...], (((1,), (1,)), ((), ())),
        preferred_element_type=jnp.float32) + b_ref[...]


def _sc_body(hp_hbm, packed_hbm, ew_hbm, out_hbm,
             packed_all, w_all, sidx_b, didx_b, rows_g, fbuf, acc,
             gsems, ssem):
    c = lax.axis_index("c")
    s = lax.axis_index("s")
    wid = c * NUM_SUBCORES + s

    def _fire_gather(ck, b):
        for g in range(CHUNK // 16):
            pv = packed_all[pl.ds(ck * CHUNK + g * 16, 16)]
            sidx_b[b, pl.ds(g * 16, 16)] = pv & IDX_MASK
            didx_b[b, pl.ds(g * 16, 16)] = lax.shift_right_logical(pv, IDX_BITS)
        pltpu.async_copy(hp_hbm.at[sidx_b.at[b]], rows_g.at[b], gsems.at[b])

    def _wait_gather(b):
        pltpu.make_async_copy(hp_hbm.at[sidx_b.at[b]], rows_g.at[b],
                              gsems.at[b]).wait()

    def _scale(ck, b):
        rows_b = rows_g.at[b]
        for g5 in range(CHUNK // 16):
            wv = w_all[pl.ds(ck * CHUNK + g5 * 16, 16)]
            for l in range(16):
                wk = wv[l]
                k = g5 * 16 + l
                for j in range(F // 32):
                    v32 = rows_b[k, pl.ds(j * 16, 16)]
                    ua = lax.bitcast_convert_type(
                        lax.shift_left(v32, 16), jnp.float32)
                    ub = lax.bitcast_convert_type(
                        v32 & jnp.int32(-65536), jnp.float32)
                    fbuf[k, pl.ds((2 * j) * 16, 16)] = ua * wk
                    fbuf[k, pl.ds((2 * j + 1) * 16, 16)] = ub * wk

    def _fire_scatter(b):
        pltpu.async_copy(fbuf, acc.at[didx_b.at[b]], ssem, add=True)

    def _wait_scatter(b):
        pltpu.make_async_copy(fbuf, acc.at[didx_b.at[b]], ssem).wait()

    pltpu.sync_copy(packed_hbm.at[pl.ds(wid * EDGES_PER_TILE, EDGES_PER_TILE)],
                    packed_all)
    pltpu.sync_copy(ew_hbm.at[pl.ds(wid * EDGES_PER_TILE, EDGES_PER_TILE)],
                    w_all)

    def _zero_body(i, _):
        z = jnp.zeros((16,), jnp.float32)
        for j in range(F // 16):
            fbuf[i, pl.ds(j * 16, 16)] = z
        return 0
    lax.fori_loop(0, CHUNK, _zero_body, 0)
    for t in range(ROWS_PER_TILE // CHUNK):
        pltpu.sync_copy(fbuf,
                        acc.at[pl.ds(s * ROWS_PER_TILE + t * CHUNK, CHUNK)])
    plsc.subcore_barrier()

    _fire_gather(0, 0)
    _fire_gather(1, 1)

    _wait_gather(0)
    _fire_gather(2, 2)
    _scale(0, 0)
    _fire_scatter(0)

    def _iter(i, _):
        for bp in range(NBUF):
            ck = NBUF * i + 1 + bp
            b = (1 + bp) % NBUF
            _wait_gather(b)
            _wait_scatter((b + NBUF - 1) % NBUF)

            @pl.when(ck + 2 < NCHUNKS)
            def _():
                _fire_gather(ck + 2, (b + 2) % NBUF)
            _scale(ck, b)
            _fire_scatter(b)
        return 0
    lax.fori_loop(0, (NCHUNKS - 2) // NBUF, _iter, 0)

    ck = NCHUNKS - 1
    b = ck % NBUF
    _wait_gather(b)
    _wait_scatter((b + NBUF - 1) % NBUF)
    _scale(ck, b)
    _fire_scatter(b)
    _wait_scatter(b)
    plsc.subcore_barrier()

    for t in range(ROWS_PER_TILE // CHUNK):
        r0 = s * ROWS_PER_TILE + t * CHUNK
        pltpu.sync_copy(acc.at[pl.ds(r0, CHUNK)], fbuf)
        pltpu.sync_copy(fbuf, out_hbm.at[c, pl.ds(r0, CHUNK)])


_sc_scatter = functools.partial(
    pl.kernel,
    mesh=plsc.VectorSubcoreMesh(core_axis_name="c", subcore_axis_name="s"),
    out_type=jax.ShapeDtypeStruct((NUM_CORES, NPAD, F), jnp.float32),
    compiler_params=pltpu.CompilerParams(use_tc_tiling_on_sc=False),
    scratch_types=[
        pltpu.VMEM((EDGES_PER_TILE,), jnp.int32),
        pltpu.VMEM((EDGES_PER_TILE,), jnp.float32),
        pltpu.VMEM((NBUF, CHUNK), jnp.int32),
        pltpu.VMEM((NBUF, CHUNK), jnp.int32),
        pltpu.VMEM((NBUF, CHUNK, F // 2), jnp.int32),
        pltpu.VMEM((CHUNK, F), jnp.float32),
        pltpu.VMEM_SHARED((NPAD, F), jnp.float32),
        pltpu.SemaphoreType.DMA((NBUF,)),
        pltpu.SemaphoreType.DMA,
    ],
)(_sc_body)


def kernel(x, edge_index, edge_weight, W_w, b_w, W_lin, b_lin):
    src = edge_index[0].astype(jnp.int32)
    dst = edge_index[1].astype(jnp.int32)
    packed = src | (dst << IDX_BITS)
    ew = edge_weight.astype(jnp.float32)

    h = pl.pallas_call(
        _mm1_kernel,
        grid=(N // M_BLK,),
        in_specs=[
            pl.BlockSpec((M_BLK, F), lambda i: (i, 0)),
            pl.BlockSpec((F, F), lambda i: (0, 0)),
            pl.BlockSpec((1, F), lambda i: (0, 0)),
        ],
        out_specs=pl.BlockSpec((M_BLK, F), lambda i: (i, 0)),
        out_shape=jax.ShapeDtypeStruct((N, F), jnp.float32),
    )(x, W_w, b_w.reshape(1, F))

    hp = lax.bitcast_convert_type(
        h.astype(jnp.bfloat16).reshape(N, F // 2, 2), jnp.int32)

    partials = _sc_scatter(hp, packed, ew)

    out = pl.pallas_call(
        _mm2_kernel,
        grid=(N // M_BLK,),
        in_specs=[
            pl.BlockSpec((NUM_CORES, M_BLK, F), lambda i: (0, i, 0)),
            pl.BlockSpec((F, F), lambda i: (0, 0)),
            pl.BlockSpec((1, F), lambda i: (0, 0)),
        ],
        out_specs=pl.BlockSpec((M_BLK, F), lambda i: (i, 0)),
        out_shape=jax.ShapeDtypeStruct((N, F), jnp.float32),
    )(partials, W_lin[:, _PERM], b_lin.reshape(1, F))
    return out
